# baseline (speedup 1.0000x reference)
import jax
import jax.numpy as jnp
from jax.experimental import pallas as pl

DIM = 256
NUM_LAYERS = 4
SE = 0.1
SC_ = 0.1
SN = 0.05
BLOCK_ROWS = 2048


def _collapse_block(h_ref, w1_ref, b1_ref, w2_ref, b2_ref, ae_ref, ac_ref,
                    an_ref, out_ref):
    h = h_ref[...]
    b1 = b1_ref[...]
    b2 = b2_ref[...]

    def _unit(x):
        n = jnp.sqrt(jnp.sum(x * x, axis=-1, keepdims=True))
        return x / jnp.maximum(n, 1e-12)

    e_dir = _unit(ae_ref[...])
    c_dir = _unit(ac_ref[...])
    n_dir = _unit(an_ref[...])

    hh = jnp.sum(h * h, axis=-1, keepdims=True)
    for _ in range(NUM_LAYERS):
        inv_hn = 1.0 / jnp.maximum(jnp.sqrt(hh), 1e-12)
        he = jnp.sum(h * e_dir, axis=-1, keepdims=True)
        hc = jnp.sum(h * c_dir, axis=-1, keepdims=True)
        hn = jnp.sum(h * n_dir, axis=-1, keepdims=True)
        ce = SE * (0.38 - he * inv_hn) * jax.lax.rsqrt(
            jnp.maximum(hh - 2.0 * he + 1.0, 1e-24))
        cc = SC_ * (0.38 - hc * inv_hn) * jax.lax.rsqrt(
            jnp.maximum(hh - 2.0 * hc + 1.0, 1e-24))
        cn = SN * (0.38 - hn * inv_hn) * jax.lax.rsqrt(
            jnp.maximum(hh - 2.0 * hn + 1.0, 1e-24))
        t = jnp.tanh(
            jax.lax.dot_general(h.astype(jnp.bfloat16), w1_ref[...],
                                (((1,), (1,)), ((), ())),
                                preferred_element_type=jnp.float32) + b1)
        delta = jax.lax.dot_general(t.astype(jnp.bfloat16), w2_ref[...],
                                    (((1,), (1,)), ((), ())),
                                    preferred_element_type=jnp.float32) + b2
        h = (h * (1.0 - ce - cc - cn) + delta
             + ce * e_dir + cc * c_dir + cn * n_dir)
        hh = jnp.sum(h * h, axis=-1, keepdims=True)
        norm = jnp.sqrt(hh)
        scale = jnp.where(norm > 10.0, 10.0 / (norm + 1e-08), 1.0)
        h = h * scale
        hh = hh * scale * scale
    out_ref[...] = h


@jax.jit
def kernel(h0, W1, b1, W2, b2, anchor_e, anchor_c, anchor_n):
    rows = h0.shape[0]
    grid = (rows // BLOCK_ROWS,)
    row_spec = pl.BlockSpec((BLOCK_ROWS, DIM), lambda i: (i, 0))
    full = pl.BlockSpec((DIM, DIM), lambda i: (0, 0))
    vec = pl.BlockSpec((1, DIM), lambda i: (0, 0))
    return pl.pallas_call(
        _collapse_block,
        grid=grid,
        in_specs=[row_spec, full, vec, full, vec, vec, vec, vec],
        out_specs=row_spec,
        out_shape=jax.ShapeDtypeStruct((rows, DIM), jnp.float32),
    )(h0, W1.astype(jnp.bfloat16), b1.reshape(1, DIM),
      W2.astype(jnp.bfloat16), b2.reshape(1, DIM),
      anchor_e.reshape(1, DIM), anchor_c.reshape(1, DIM),
      anchor_n.reshape(1, DIM))


# speedup vs baseline: 1.3796x; 1.3796x over previous
import jax
import jax.numpy as jnp
from jax.experimental import pallas as pl

DIM = 256
NUM_LAYERS = 4
SE = 0.1
SC_ = 0.1
SN = 0.05
BLOCK_ROWS = 2048


def _collapse_block(h_ref, w1_ref, b1_ref, w2_ref, b2_ref, anch_ref,
                    out_ref):
    h = h_ref[...]
    b1 = b1_ref[...]
    b2 = b2_ref[...]

    anch = anch_ref[...]
    anorm = jnp.sqrt(jnp.sum(anch * anch, axis=-1, keepdims=True))
    dirs = anch / jnp.maximum(anorm, 1e-12)
    lane = jax.lax.broadcasted_iota(jnp.int32, (1, 3), 1)
    svec = jnp.where(lane == 2, SN, jnp.where(lane == 0, SE, SC_))

    hh = jnp.sum(h * h, axis=-1, keepdims=True)
    for _ in range(NUM_LAYERS):
        inv_hn = 1.0 / jnp.maximum(jnp.sqrt(hh), 1e-12)
        au = jax.lax.dot_general(h, dirs, (((1,), (1,)), ((), ())),
                                 preferred_element_type=jnp.float32)
        c3 = svec * (0.38 - au * inv_hn) * jax.lax.rsqrt(
            jnp.maximum(hh - 2.0 * au + 1.0, 1e-24))
        t = jnp.tanh(
            jax.lax.dot_general(h, w1_ref[...], (((1,), (1,)), ((), ())),
                                preferred_element_type=jnp.float32) + b1)
        delta = jax.lax.dot_general(t, w2_ref[...], (((1,), (1,)), ((), ())),
                                    preferred_element_type=jnp.float32) + b2
        adds = jax.lax.dot_general(c3, dirs, (((1,), (0,)), ((), ())),
                                   preferred_element_type=jnp.float32)
        csum = jnp.sum(c3, axis=-1, keepdims=True)
        h = h * (1.0 - csum) + delta + adds
        hh = jnp.sum(h * h, axis=-1, keepdims=True)
        norm = jnp.sqrt(hh)
        scale = jnp.where(norm > 10.0, 10.0 / (norm + 1e-08), 1.0)
        h = h * scale
        hh = hh * scale * scale
    out_ref[...] = h


@jax.jit
def kernel(h0, W1, b1, W2, b2, anchor_e, anchor_c, anchor_n):
    rows = h0.shape[0]
    grid = (rows // BLOCK_ROWS,)
    row_spec = pl.BlockSpec((BLOCK_ROWS, DIM), lambda i: (i, 0))
    full = pl.BlockSpec((DIM, DIM), lambda i: (0, 0))
    vec = pl.BlockSpec((1, DIM), lambda i: (0, 0))
    anch_spec = pl.BlockSpec((3, DIM), lambda i: (0, 0))
    anchors = jnp.stack([anchor_e, anchor_c, anchor_n], axis=0)
    return pl.pallas_call(
        _collapse_block,
        grid=grid,
        in_specs=[row_spec, full, vec, full, vec, anch_spec],
        out_specs=row_spec,
        out_shape=jax.ShapeDtypeStruct((rows, DIM), jnp.float32),
    )(h0, W1, b1.reshape(1, DIM), W2, b2.reshape(1, DIM), anchors)


# speedup vs baseline: 1.4158x; 1.0263x over previous
import jax
import jax.numpy as jnp
from jax.experimental import pallas as pl

DIM = 256
NUM_LAYERS = 4
SE = 0.1
SC_ = 0.1
SN = 0.05
BLOCK_ROWS = 4096


def _collapse_block(h_ref, w1_ref, b1_ref, w2_ref, b2_ref, anch_ref,
                    out_ref):
    h = h_ref[...]
    b1 = b1_ref[...]
    b2 = b2_ref[...]

    anch = anch_ref[...]
    anorm = jnp.sqrt(jnp.sum(anch * anch, axis=-1, keepdims=True))
    dirs = anch / jnp.maximum(anorm, 1e-12)
    lane = jax.lax.broadcasted_iota(jnp.int32, (1, 3), 1)
    svec = jnp.where(lane == 2, SN, jnp.where(lane == 0, SE, SC_))

    hh = jnp.sum(h * h, axis=-1, keepdims=True)
    for _ in range(NUM_LAYERS):
        inv_hn = 1.0 / jnp.maximum(jnp.sqrt(hh), 1e-12)
        au = jax.lax.dot_general(h, dirs, (((1,), (1,)), ((), ())),
                                 preferred_element_type=jnp.float32)
        c3 = svec * (0.38 - au * inv_hn) * jax.lax.rsqrt(
            jnp.maximum(hh - 2.0 * au + 1.0, 1e-24))
        t = jnp.tanh(
            jax.lax.dot_general(h, w1_ref[...], (((1,), (1,)), ((), ())),
                                preferred_element_type=jnp.float32) + b1)
        delta = jax.lax.dot_general(t, w2_ref[...], (((1,), (1,)), ((), ())),
                                    preferred_element_type=jnp.float32) + b2
        adds = jax.lax.dot_general(c3, dirs, (((1,), (0,)), ((), ())),
                                   preferred_element_type=jnp.float32)
        csum = jnp.sum(c3, axis=-1, keepdims=True)
        h = h * (1.0 - csum) + delta + adds
        hh = jnp.sum(h * h, axis=-1, keepdims=True)
        norm = jnp.sqrt(hh)
        scale = jnp.where(norm > 10.0, 10.0 / (norm + 1e-08), 1.0)
        h = h * scale
        hh = hh * scale * scale
    out_ref[...] = h


@jax.jit
def kernel(h0, W1, b1, W2, b2, anchor_e, anchor_c, anchor_n):
    rows = h0.shape[0]
    grid = (rows // BLOCK_ROWS,)
    row_spec = pl.BlockSpec((BLOCK_ROWS, DIM), lambda i: (i, 0))
    full = pl.BlockSpec((DIM, DIM), lambda i: (0, 0))
    vec = pl.BlockSpec((1, DIM), lambda i: (0, 0))
    anch_spec = pl.BlockSpec((3, DIM), lambda i: (0, 0))
    anchors = jnp.stack([anchor_e, anchor_c, anchor_n], axis=0)
    return pl.pallas_call(
        _collapse_block,
        grid=grid,
        in_specs=[row_spec, full, vec, full, vec, anch_spec],
        out_specs=row_spec,
        out_shape=jax.ShapeDtypeStruct((rows, DIM), jnp.float32),
    )(h0, W1, b1.reshape(1, DIM), W2, b2.reshape(1, DIM), anchors)


# speedup vs baseline: 1.5857x; 1.1200x over previous
import jax
import jax.numpy as jnp
from jax.experimental import pallas as pl

DIM = 256
NUM_LAYERS = 4
SE = 0.1
SC_ = 0.1
SN = 0.05
BLOCK_ROWS = 4096


def _collapse_block(h_ref, w1_ref, b1_ref, w2_ref, b2_ref, anch_ref,
                    out_ref):
    g = h_ref[...]
    b1 = b1_ref[...]

    anch = anch_ref[...]
    anorm = jnp.sqrt(jnp.sum(anch * anch, axis=-1, keepdims=True))
    dirs = anch / jnp.maximum(anorm, 1e-12)
    mat4 = jnp.concatenate([dirs, b2_ref[...]], axis=0)
    lane = jax.lax.broadcasted_iota(jnp.int32, (1, 3), 1)
    svec = jnp.where(lane == 2, SN, jnp.where(lane == 0, SE, SC_))

    hh = jnp.sum(g * g, axis=-1, keepdims=True)
    s = None
    for _ in range(NUM_LAYERS):
        inv_hn = jax.lax.rsqrt(jnp.maximum(hh, 1e-24))
        au_g = jax.lax.dot_general(g, dirs, (((1,), (1,)), ((), ())),
                                   preferred_element_type=jnp.float32)
        au = au_g if s is None else au_g * s
        c3 = svec * (0.38 - au * inv_hn) * jax.lax.rsqrt(
            jnp.maximum(hh - 2.0 * au + 1.0, 1e-24))
        gw1 = jax.lax.dot_general(g, w1_ref[...], (((1,), (1,)), ((), ())),
                                  preferred_element_type=jnp.float32)
        t = jnp.tanh((gw1 if s is None else s * gw1) + b1)
        delta = jax.lax.dot_general(t, w2_ref[...], (((1,), (1,)), ((), ())),
                                    preferred_element_type=jnp.float32)
        c4 = jnp.concatenate([c3, jnp.ones_like(hh)], axis=1)
        adds = jax.lax.dot_general(c4, mat4, (((1,), (0,)), ((), ())),
                                   preferred_element_type=jnp.float32)
        csum = jnp.sum(c3, axis=-1, keepdims=True)
        m = (1.0 - csum) if s is None else s * (1.0 - csum)
        g = g * m + delta + adds
        hh = jnp.sum(g * g, axis=-1, keepdims=True)
        norm = jnp.sqrt(hh)
        s = jnp.where(norm > 10.0, 10.0 / (norm + 1e-08), 1.0)
        hh = hh * s * s
    out_ref[...] = g * s


@jax.jit
def kernel(h0, W1, b1, W2, b2, anchor_e, anchor_c, anchor_n):
    rows = h0.shape[0]
    grid = (rows // BLOCK_ROWS,)
    row_spec = pl.BlockSpec((BLOCK_ROWS, DIM), lambda i: (i, 0))
    full = pl.BlockSpec((DIM, DIM), lambda i: (0, 0))
    vec = pl.BlockSpec((1, DIM), lambda i: (0, 0))
    anch_spec = pl.BlockSpec((3, DIM), lambda i: (0, 0))
    anchors = jnp.stack([anchor_e, anchor_c, anchor_n], axis=0)
    return pl.pallas_call(
        _collapse_block,
        grid=grid,
        in_specs=[row_spec, full, vec, full, vec, anch_spec],
        out_specs=row_spec,
        out_shape=jax.ShapeDtypeStruct((rows, DIM), jnp.float32),
    )(h0, W1, b1.reshape(1, DIM), W2, b2.reshape(1, DIM), anchors)


# speedup vs baseline: 1.6283x; 1.0269x over previous
import jax
import jax.numpy as jnp
from jax.experimental import pallas as pl

DIM = 256
NUM_LAYERS = 4
SE = 0.1
SC_ = 0.1
SN = 0.05
BLOCK_ROWS = 4096


def _collapse_block(h_ref, w1_ref, b1_ref, w2_ref, b2_ref, anch_ref,
                    out_ref):
    g = h_ref[...]
    b1 = b1_ref[...]

    anch = anch_ref[...]
    anorm = jnp.sqrt(jnp.sum(anch * anch, axis=-1, keepdims=True))
    dirs = anch / jnp.maximum(anorm, 1e-12)
    mat4 = jnp.concatenate([dirs, b2_ref[...]], axis=0)
    lane = jax.lax.broadcasted_iota(jnp.int32, (1, 3), 1)
    svec = jnp.where(lane == 2, SN, jnp.where(lane == 0, SE, SC_))

    hh = jnp.sum(g * g, axis=-1, keepdims=True)
    s = None
    for _ in range(NUM_LAYERS):
        inv_hn = jax.lax.rsqrt(jnp.maximum(hh, 1e-24))
        au_g = jax.lax.dot_general(g, dirs, (((1,), (1,)), ((), ())),
                                   preferred_element_type=jnp.float32)
        au = au_g if s is None else au_g * s
        c3 = svec * (0.38 - au * inv_hn) * jax.lax.rsqrt(
            jnp.maximum(hh - 2.0 * au + 1.0, 1e-24))
        gw1 = jax.lax.dot_general(g.astype(jnp.bfloat16), w1_ref[...],
                                  (((1,), (1,)), ((), ())),
                                  preferred_element_type=jnp.float32)
        t = jnp.tanh((gw1 if s is None else s * gw1) + b1)
        delta = jax.lax.dot_general(t.astype(jnp.bfloat16), w2_ref[...],
                                    (((1,), (1,)), ((), ())),
                                    preferred_element_type=jnp.float32)
        c4 = jnp.concatenate([c3, jnp.ones_like(hh)], axis=1)
        adds = jax.lax.dot_general(c4, mat4, (((1,), (0,)), ((), ())),
                                   preferred_element_type=jnp.float32)
        csum = jnp.sum(c3, axis=-1, keepdims=True)
        m = (1.0 - csum) if s is None else s * (1.0 - csum)
        g = g * m + delta + adds
        hh = jnp.sum(g * g, axis=-1, keepdims=True)
        norm = jnp.sqrt(hh)
        s = jnp.where(norm > 10.0, 10.0 / (norm + 1e-08), 1.0)
        hh = hh * s * s
    out_ref[...] = g * s


@jax.jit
def kernel(h0, W1, b1, W2, b2, anchor_e, anchor_c, anchor_n):
    rows = h0.shape[0]
    grid = (rows // BLOCK_ROWS,)
    row_spec = pl.BlockSpec((BLOCK_ROWS, DIM), lambda i: (i, 0))
    full = pl.BlockSpec((DIM, DIM), lambda i: (0, 0))
    vec = pl.BlockSpec((1, DIM), lambda i: (0, 0))
    anch_spec = pl.BlockSpec((3, DIM), lambda i: (0, 0))
    anchors = jnp.stack([anchor_e, anchor_c, anchor_n], axis=0)
    return pl.pallas_call(
        _collapse_block,
        grid=grid,
        in_specs=[row_spec, full, vec, full, vec, anch_spec],
        out_specs=row_spec,
        out_shape=jax.ShapeDtypeStruct((rows, DIM), jnp.float32),
    )(h0, W1.astype(jnp.bfloat16), b1.reshape(1, DIM),
      W2.astype(jnp.bfloat16), b2.reshape(1, DIM), anchors)
